# baseline (speedup 1.0000x reference)
import functools

import jax
import jax.numpy as jnp
from jax import lax
from jax.experimental import pallas as pl
from jax.experimental.pallas import tpu as pltpu
from jax.experimental.pallas import tpu_sc as plsc

N = 10000
E = 320000
F = 128
H = 128
C = 40
CP = 48
NC, NS = 2, 16
NW = NC * NS
EPW = E // NW
CHUNK = 125
CPW = EPW // CHUNK
NPAD = 10240
STRIPE = NPAD // NS
_STRIPE_CHUNKS = [(t * 80, 80) for t in range(STRIPE // 80)]

_mesh = plsc.VectorSubcoreMesh(core_axis_name="c", subcore_axis_name="s")


@functools.partial(
    pl.kernel,
    out_type=jax.ShapeDtypeStruct((NW, N), jnp.float32),
    mesh=_mesh,
    scratch_types=[
        pltpu.VMEM((N,), jnp.float32),
        pltpu.VMEM((EPW // 16, 16), jnp.int32),
    ],
    compiler_params=pltpu.CompilerParams(needs_layout_passes=False,
                                        use_tc_tiling_on_sc=False),
)
def _sc_degree(col_hbm, z_hbm, out_hbm, hist, colv):
    c = lax.axis_index("c")
    s = lax.axis_index("s")
    w = c * NS + s
    pltpu.sync_copy(z_hbm, hist)
    pltpu.sync_copy(col_hbm.at[w], colv)
    ones = jnp.full((16,), 1.0, jnp.float32)

    @pl.loop(0, EPW // 16)
    def _(j):
        plsc.addupdate_scatter(hist, [colv[j]], ones)

    pltpu.sync_copy(hist, out_hbm.at[w])


def _make_prop(d, tc_tiling, nep):
    cpe = CPW // nep

    @functools.partial(
        pl.kernel,
        out_type=jax.ShapeDtypeStruct((NC, NPAD, d), jnp.float32),
        mesh=_mesh,
        scratch_types=[
            pltpu.VMEM_SHARED((NPAD, d), jnp.float32),
            pltpu.VMEM((cpe, CHUNK), jnp.int32),
            pltpu.VMEM((cpe, CHUNK), jnp.int32),
            pltpu.VMEM((CHUNK, d), jnp.float32),
            pltpu.VMEM((CHUNK, d), jnp.float32),
            pltpu.SemaphoreType.DMA,
            pltpu.SemaphoreType.DMA,
        ],
        compiler_params=pltpu.CompilerParams(use_tc_tiling_on_sc=tc_tiling),
    )
    def prop(hs_hbm, e_hbm, z_hbm, out_hbm,
             acc, rowv, colv, buf0, buf1, sem0, sem1):
        c = lax.axis_index("c")
        s = lax.axis_index("s")
        w = c * NS + s
        base = s * STRIPE
        pltpu.sync_copy(z_hbm, buf0)
        for off, sz in _STRIPE_CHUNKS:
            pltpu.sync_copy(buf0.at[pl.ds(0, sz)],
                            acc.at[pl.ds(base + off, sz)])
        plsc.subcore_barrier()

        for ep in range(nep):
            pltpu.sync_copy(e_hbm.at[0, w, pl.ds(ep * cpe, cpe)], rowv)
            pltpu.sync_copy(e_hbm.at[1, w, pl.ds(ep * cpe, cpe)], colv)

            pltpu.async_copy(hs_hbm.at[rowv.at[0]], buf0, sem0)

            @pl.loop(0, cpe // 2)
            def _(g):
                j0 = 2 * g
                a1 = pltpu.async_copy(hs_hbm.at[rowv.at[j0 + 1]], buf1, sem1)
                pltpu.make_async_copy(hs_hbm.at[rowv.at[j0]], buf0, sem0).wait()
                pltpu.sync_copy(buf0, acc.at[colv.at[j0]], add=True)

                @pl.when(g + 1 < cpe // 2)
                def _():
                    pltpu.async_copy(hs_hbm.at[rowv.at[j0 + 2]], buf0, sem0)

                a1.wait()
                pltpu.sync_copy(buf1, acc.at[colv.at[j0 + 1]], add=True)

        plsc.subcore_barrier()
        for off, sz in _STRIPE_CHUNKS:
            pltpu.sync_copy(acc.at[pl.ds(base + off, sz)],
                            buf0.at[pl.ds(0, sz)])
            pltpu.sync_copy(buf0.at[pl.ds(0, sz)],
                            out_hbm.at[c, pl.ds(base + off, sz)])

    return prop


_prop_h = _make_prop(H, False, 2)
_prop_c = _make_prop(CP, False, 1)


NB = 1000


def _dis(deg_ref):
    return lax.rsqrt(jnp.sum(deg_ref[...], axis=0)[:, None] + 1.0)


def _tc1_body(x_ref, w_ref, deg_ref, o_ref, dis_ref):
    dis = _dis(deg_ref)
    h = jnp.dot(x_ref[...], w_ref[...], preferred_element_type=jnp.float32)
    o_ref[...] = h * dis
    dis_ref[...] = jnp.broadcast_to(dis, (N, 8))


def _tc2_body(p_ref, hs1_ref, dis_ref, b1_ref, w2_ref, o_ref):
    dis = dis_ref[...][:, 0:1]
    p = p_ref[0] + p_ref[1] + hs1_ref[...]
    z = jnp.maximum(p * dis + b1_ref[...], 0.0)
    o_ref[...] = jnp.dot(z, w2_ref[...],
                         preferred_element_type=jnp.float32) * dis


def _tc3_body(q_ref, hs2_ref, dis_ref, b2_ref, o_ref):
    p2 = q_ref[0] + q_ref[1] + hs2_ref[...]
    o_ref[...] = (p2 * dis_ref[...][:, 0:1])[:, :C] + b2_ref[...]


def kernel(x, edge_index, W1, b1, W2, b2):
    e4 = edge_index.reshape(2, NW, CPW, CHUNK)
    col16 = edge_index[1].reshape(NW, EPW // 16, 16)
    w2p = jnp.pad(W2, ((0, 0), (0, CP - C)))
    b1r = b1.reshape(1, H)
    b2r = b2.reshape(1, C)
    zdeg = jnp.zeros((N,), jnp.float32)
    zh = jnp.zeros((CHUNK, H), jnp.float32)
    zc = jnp.zeros((CHUNK, CP), jnp.float32)

    degp = _sc_degree(col16, zdeg)

    hs1, dis8 = pl.pallas_call(
        _tc1_body,
        out_shape=(jax.ShapeDtypeStruct((N, H), jnp.float32),
                   jax.ShapeDtypeStruct((N, 8), jnp.float32)),
    )(x, W1, degp)

    P = _prop_h(hs1, e4, zh)

    hs2 = pl.pallas_call(
        _tc2_body,
        grid=(N // NB,),
        in_specs=[
            pl.BlockSpec((NC, NB, H), lambda i: (0, i, 0)),
            pl.BlockSpec((NB, H), lambda i: (i, 0)),
            pl.BlockSpec((NB, 8), lambda i: (i, 0)),
            pl.BlockSpec((1, H), lambda i: (0, 0)),
            pl.BlockSpec((H, CP), lambda i: (0, 0)),
        ],
        out_specs=pl.BlockSpec((NB, CP), lambda i: (i, 0)),
        out_shape=jax.ShapeDtypeStruct((N, CP), jnp.float32),
    )(P, hs1, dis8, b1r, w2p)

    Q = _prop_c(hs2, e4, zc)

    y = pl.pallas_call(
        _tc3_body,
        grid=(N // NB,),
        in_specs=[
            pl.BlockSpec((NC, NB, CP), lambda i: (0, i, 0)),
            pl.BlockSpec((NB, CP), lambda i: (i, 0)),
            pl.BlockSpec((NB, 8), lambda i: (i, 0)),
            pl.BlockSpec((1, C), lambda i: (0, 0)),
        ],
        out_specs=pl.BlockSpec((NB, C), lambda i: (i, 0)),
        out_shape=jax.ShapeDtypeStruct((N, C), jnp.float32),
    )(Q, hs2, dis8, b2r)

    return y


# speedup vs baseline: 1.0329x; 1.0329x over previous
import functools

import jax
import jax.numpy as jnp
from jax import lax
from jax.experimental import pallas as pl
from jax.experimental.pallas import tpu as pltpu
from jax.experimental.pallas import tpu_sc as plsc

N = 10000
E = 320000
F = 128
H = 128
C = 40
CP = 48
NC, NS = 2, 16
NW = NC * NS
EPW = E // NW
CHUNK = 125
CPW = EPW // CHUNK
NPAD = 10240
STRIPE = NPAD // NS
_STRIPE_CHUNKS = [(t * 80, 80) for t in range(STRIPE // 80)]

_mesh = plsc.VectorSubcoreMesh(core_axis_name="c", subcore_axis_name="s")


@functools.partial(
    pl.kernel,
    out_type=jax.ShapeDtypeStruct((NW, N), jnp.float32),
    mesh=_mesh,
    scratch_types=[
        pltpu.VMEM((N,), jnp.float32),
        pltpu.VMEM((EPW // 16, 16), jnp.int32),
    ],
    compiler_params=pltpu.CompilerParams(needs_layout_passes=False,
                                        use_tc_tiling_on_sc=False),
)
def _sc_degree(col_hbm, z_hbm, out_hbm, hist, colv):
    c = lax.axis_index("c")
    s = lax.axis_index("s")
    w = c * NS + s
    pltpu.sync_copy(z_hbm, hist)
    pltpu.sync_copy(col_hbm.at[w], colv)
    ones = jnp.full((16,), 1.0, jnp.float32)

    @pl.loop(0, EPW // 16)
    def _(j):
        plsc.addupdate_scatter(hist, [colv[j]], ones)

    pltpu.sync_copy(hist, out_hbm.at[w])


def _make_prop(d, tc_tiling, nep):
    cpe = CPW // nep

    @functools.partial(
        pl.kernel,
        out_type=jax.ShapeDtypeStruct((NC, NPAD, d), jnp.float32),
        mesh=_mesh,
        scratch_types=[
            pltpu.VMEM_SHARED((NPAD, d), jnp.float32),
            pltpu.VMEM((cpe, CHUNK), jnp.int32),
            pltpu.VMEM((cpe, CHUNK), jnp.int32),
            pltpu.VMEM((CHUNK, d), jnp.float32),
            pltpu.VMEM((CHUNK, d), jnp.float32),
            pltpu.SemaphoreType.DMA,
            pltpu.SemaphoreType.DMA,
        ],
        compiler_params=pltpu.CompilerParams(use_tc_tiling_on_sc=tc_tiling),
    )
    def prop(hs_hbm, e_hbm, z_hbm, out_hbm,
             acc, rowv, colv, buf0, buf1, sem0, sem1):
        c = lax.axis_index("c")
        s = lax.axis_index("s")
        w = c * NS + s
        base = s * STRIPE
        pltpu.sync_copy(z_hbm, buf0)
        for off, sz in _STRIPE_CHUNKS:
            pltpu.sync_copy(buf0.at[pl.ds(0, sz)],
                            acc.at[pl.ds(base + off, sz)])
        plsc.subcore_barrier()

        for ep in range(nep):
            pltpu.sync_copy(e_hbm.at[0, w, pl.ds(ep * cpe, cpe)], rowv)
            pltpu.sync_copy(e_hbm.at[1, w, pl.ds(ep * cpe, cpe)], colv)

            pltpu.async_copy(hs_hbm.at[rowv.at[0]], buf0, sem0)

            @pl.loop(0, cpe // 2)
            def _(g):
                j0 = 2 * g
                a1 = pltpu.async_copy(hs_hbm.at[rowv.at[j0 + 1]], buf1, sem1)
                pltpu.make_async_copy(hs_hbm.at[rowv.at[j0]], buf0, sem0).wait()
                pltpu.sync_copy(buf0, acc.at[colv.at[j0]], add=True)

                @pl.when(g + 1 < cpe // 2)
                def _():
                    pltpu.async_copy(hs_hbm.at[rowv.at[j0 + 2]], buf0, sem0)

                a1.wait()
                pltpu.sync_copy(buf1, acc.at[colv.at[j0 + 1]], add=True)

        plsc.subcore_barrier()
        for off, sz in _STRIPE_CHUNKS:
            pltpu.sync_copy(acc.at[pl.ds(base + off, sz)],
                            buf0.at[pl.ds(0, sz)])
            pltpu.sync_copy(buf0.at[pl.ds(0, sz)],
                            out_hbm.at[c, pl.ds(base + off, sz)])

    return prop


_prop_h = _make_prop(H, False, 2)
_prop_c = _make_prop(CP, False, 1)


NB = 1000


def _dis(deg_ref):
    return lax.rsqrt(jnp.sum(deg_ref[...], axis=0)[:, None] + 1.0)


def _tc1_body(x_ref, w_ref, deg_ref, o_ref):
    h = jnp.dot(x_ref[...], w_ref[...], preferred_element_type=jnp.float32)
    o_ref[...] = h * _dis(deg_ref)


def _tc2_body(p_ref, hs1_ref, deg_ref, b1_ref, w2_ref, o_ref):
    dis = _dis(deg_ref)
    p = p_ref[0, :N, :] + p_ref[1, :N, :] + hs1_ref[...]
    z = jnp.maximum(p * dis + b1_ref[...], 0.0)
    o_ref[...] = jnp.dot(z, w2_ref[...],
                         preferred_element_type=jnp.float32) * dis


def _tc3_body(q_ref, hs2_ref, deg_ref, b2_ref, o_ref):
    p2 = q_ref[0, :N, :] + q_ref[1, :N, :] + hs2_ref[...]
    o_ref[...] = (p2 * _dis(deg_ref))[:, :C] + b2_ref[...]


def kernel(x, edge_index, W1, b1, W2, b2):
    e4 = edge_index.reshape(2, NW, CPW, CHUNK)
    col16 = edge_index[1].reshape(NW, EPW // 16, 16)
    w2p = jnp.pad(W2, ((0, 0), (0, CP - C)))
    b1r = b1.reshape(1, H)
    b2r = b2.reshape(1, C)
    zdeg = jnp.zeros((N,), jnp.float32)
    zh = jnp.zeros((CHUNK, H), jnp.float32)
    zc = jnp.zeros((CHUNK, CP), jnp.float32)

    degp = _sc_degree(col16, zdeg)

    hs1 = pl.pallas_call(
        _tc1_body,
        out_shape=jax.ShapeDtypeStruct((N, H), jnp.float32),
    )(x, W1, degp)

    P = _prop_h(hs1, e4, zh)

    hs2 = pl.pallas_call(
        _tc2_body,
        out_shape=jax.ShapeDtypeStruct((N, CP), jnp.float32),
    )(P, hs1, degp, b1r, w2p)

    Q = _prop_c(hs2, e4, zc)

    y = pl.pallas_call(
        _tc3_body,
        out_shape=jax.ShapeDtypeStruct((N, C), jnp.float32),
    )(Q, hs2, degp, b2r)

    return y


# speedup vs baseline: 1.0791x; 1.0448x over previous
import functools

import jax
import jax.numpy as jnp
from jax import lax
from jax.experimental import pallas as pl
from jax.experimental.pallas import tpu as pltpu
from jax.experimental.pallas import tpu_sc as plsc

N = 10000
E = 320000
F = 128
H = 128
C = 40
CP = 48
NC, NS = 2, 16
NW = NC * NS
EPW = E // NW
CHUNK = 125
CPW = EPW // CHUNK
NPAD = 10240
STRIPE = NPAD // NS
_STRIPE_CHUNKS = [(t * 80, 80) for t in range(STRIPE // 80)]

_mesh = plsc.VectorSubcoreMesh(core_axis_name="c", subcore_axis_name="s")


@functools.partial(
    pl.kernel,
    out_type=jax.ShapeDtypeStruct((NW, N), jnp.float32),
    mesh=_mesh,
    scratch_types=[
        pltpu.VMEM((N,), jnp.float32),
        pltpu.VMEM((EPW // 16, 16), jnp.int32),
    ],
    compiler_params=pltpu.CompilerParams(needs_layout_passes=False,
                                        use_tc_tiling_on_sc=False),
)
def _sc_degree(col_hbm, z_hbm, out_hbm, hist, colv):
    c = lax.axis_index("c")
    s = lax.axis_index("s")
    w = c * NS + s
    pltpu.sync_copy(z_hbm, hist)
    pltpu.sync_copy(col_hbm.at[w], colv)
    ones = jnp.full((16,), 1.0, jnp.float32)

    @pl.loop(0, EPW // 16)
    def _(j):
        plsc.addupdate_scatter(hist, [colv[j]], ones)

    pltpu.sync_copy(hist, out_hbm.at[w])


def _make_prop(d, tc_tiling, nep):
    cpe = CPW // nep

    @functools.partial(
        pl.kernel,
        out_type=jax.ShapeDtypeStruct((NC, NPAD, d), jnp.float32),
        mesh=_mesh,
        scratch_types=[
            pltpu.VMEM_SHARED((NPAD, d), jnp.float32),
            pltpu.VMEM((cpe, CHUNK), jnp.int32),
            pltpu.VMEM((cpe, CHUNK), jnp.int32),
            pltpu.VMEM((CHUNK, d), jnp.float32),
            pltpu.VMEM((CHUNK, d), jnp.float32),
            pltpu.SemaphoreType.DMA,
            pltpu.SemaphoreType.DMA,
        ],
        compiler_params=pltpu.CompilerParams(use_tc_tiling_on_sc=tc_tiling),
    )
    def prop(hs_hbm, row_hbm, col_hbm, z_hbm, out_hbm,
             acc, rowv, colv, buf0, buf1, sem0, sem1):
        c = lax.axis_index("c")
        s = lax.axis_index("s")
        w = c * NS + s
        base = s * STRIPE
        pltpu.sync_copy(z_hbm, buf0)
        for off, sz in _STRIPE_CHUNKS:
            pltpu.sync_copy(buf0.at[pl.ds(0, sz)],
                            acc.at[pl.ds(base + off, sz)])
        plsc.subcore_barrier()

        for ep in range(nep):
            pltpu.sync_copy(row_hbm.at[w, pl.ds(ep * cpe, cpe)], rowv)
            pltpu.sync_copy(col_hbm.at[w, pl.ds(ep * cpe, cpe)], colv)

            pltpu.async_copy(hs_hbm.at[rowv.at[0]], buf0, sem0)

            @pl.loop(0, cpe // 2)
            def _(g):
                j0 = 2 * g
                a1 = pltpu.async_copy(hs_hbm.at[rowv.at[j0 + 1]], buf1, sem1)
                pltpu.make_async_copy(hs_hbm.at[rowv.at[j0]], buf0, sem0).wait()
                pltpu.sync_copy(buf0, acc.at[colv.at[j0]], add=True)

                @pl.when(g + 1 < cpe // 2)
                def _():
                    pltpu.async_copy(hs_hbm.at[rowv.at[j0 + 2]], buf0, sem0)

                a1.wait()
                pltpu.sync_copy(buf1, acc.at[colv.at[j0 + 1]], add=True)

        plsc.subcore_barrier()
        for off, sz in _STRIPE_CHUNKS:
            pltpu.sync_copy(acc.at[pl.ds(base + off, sz)],
                            buf0.at[pl.ds(0, sz)])
            pltpu.sync_copy(buf0.at[pl.ds(0, sz)],
                            out_hbm.at[c, pl.ds(base + off, sz)])

    return prop


_prop_h = _make_prop(H, False, 2)
_prop_c = _make_prop(CP, False, 1)


NB = 1000


def _tc0_body(e_ref, row_ref, col_ref):
    v = e_ref[...]
    row_ref[...] = v[0]
    col_ref[...] = v[1]


def _dis(deg_ref):
    return lax.rsqrt(jnp.sum(deg_ref[...], axis=0)[:, None] + 1.0)


def _tc1_body(x_ref, w_ref, deg_ref, o_ref):
    h = jnp.dot(x_ref[...], w_ref[...], preferred_element_type=jnp.float32)
    o_ref[...] = h * _dis(deg_ref)


def _tc2_body(p_ref, hs1_ref, deg_ref, b1_ref, w2_ref, o_ref):
    dis = _dis(deg_ref)
    p = p_ref[0, :N, :] + p_ref[1, :N, :] + hs1_ref[...]
    z = jnp.maximum(p * dis + b1_ref[...], 0.0)
    o_ref[...] = jnp.dot(z, w2_ref[...],
                         preferred_element_type=jnp.float32) * dis


def _tc3_body(q_ref, hs2_ref, deg_ref, b2_ref, o_ref):
    p2 = q_ref[0, :N, :] + q_ref[1, :N, :] + hs2_ref[...]
    o_ref[...] = (p2 * _dis(deg_ref))[:, :C] + b2_ref[...]


def kernel(x, edge_index, W1, b1, W2, b2):
    rowd, cold = pl.pallas_call(
        _tc0_body,
        out_shape=(jax.ShapeDtypeStruct((E,), jnp.int32),
                   jax.ShapeDtypeStruct((E,), jnp.int32)),
    )(edge_index)
    row3 = rowd.reshape(NW, CPW, CHUNK)
    col3 = cold.reshape(NW, CPW, CHUNK)
    col16 = cold.reshape(NW, EPW // 16, 16)
    w2p = jnp.pad(W2, ((0, 0), (0, CP - C)))
    b1r = b1.reshape(1, H)
    b2r = b2.reshape(1, C)
    zdeg = jnp.zeros((N,), jnp.float32)
    zh = jnp.zeros((CHUNK, H), jnp.float32)
    zc = jnp.zeros((CHUNK, CP), jnp.float32)

    degp = _sc_degree(col16, zdeg)

    hs1 = pl.pallas_call(
        _tc1_body,
        out_shape=jax.ShapeDtypeStruct((N, H), jnp.float32),
    )(x, W1, degp)

    P = _prop_h(hs1, row3, col3, zh)

    hs2 = pl.pallas_call(
        _tc2_body,
        out_shape=jax.ShapeDtypeStruct((N, CP), jnp.float32),
    )(P, hs1, degp, b1r, w2p)

    Q = _prop_c(hs2, row3, col3, zc)

    y = pl.pallas_call(
        _tc3_body,
        out_shape=jax.ShapeDtypeStruct((N, C), jnp.float32),
    )(Q, hs2, degp, b2r)

    return y
